# Initial kernel scaffold; baseline (speedup 1.0000x reference)
import jax
import jax.numpy as jnp
from jax.experimental import pallas as pl


def kernel(h_pooled, W, b, hard):
    raise NotImplementedError("write your pallas kernel here")


# speedup vs baseline: 3.8774x; 3.8774x over previous
import functools

import jax
import jax.numpy as jnp
from jax.experimental import pallas as pl
from jax.experimental.pallas import tpu as pltpu

BATCH = 16384
HIDDEN = 4096
NUM_LAYERS = 32
TOPK = 4
BLOCK_R = 1024


def _router_kernel(hard_ref, h_ref, w_ref, b_ref, out_ref):
    logits = jax.lax.dot_general(
        h_ref[...], w_ref[...],
        dimension_numbers=(((1,), (1,)), ((), ())),
        preferred_element_type=jnp.float32,
    ) + b_ref[...]
    sig = jax.nn.sigmoid(logits)

    cols = jax.lax.broadcasted_iota(jnp.int32, sig.shape, 1)
    s = sig
    gates = jnp.zeros_like(sig)
    for _ in range(TOPK):
        m = jnp.max(s, axis=1, keepdims=True)
        ismax = s == m
        first = jnp.min(jnp.where(ismax, cols, NUM_LAYERS), axis=1,
                        keepdims=True)
        sel = ismax & (cols == first)
        gates = jnp.where(sel, 1.0, gates)
        s = jnp.where(sel, -jnp.inf, s)

    out_ref[...] = jnp.where(hard_ref[0] != 0, gates, sig)


def kernel(h_pooled, W, b, hard):
    hard_arr = jnp.asarray(hard, dtype=jnp.int32).reshape((1,))
    b2 = b.reshape(1, NUM_LAYERS)
    grid = (BATCH // BLOCK_R,)
    return pl.pallas_call(
        _router_kernel,
        grid_spec=pltpu.PrefetchScalarGridSpec(
            num_scalar_prefetch=1,
            grid=grid,
            in_specs=[
                pl.BlockSpec((BLOCK_R, HIDDEN), lambda i, *_: (i, 0)),
                pl.BlockSpec((NUM_LAYERS, HIDDEN), lambda i, *_: (0, 0)),
                pl.BlockSpec((1, NUM_LAYERS), lambda i, *_: (0, 0)),
            ],
            out_specs=pl.BlockSpec((BLOCK_R, NUM_LAYERS), lambda i, *_: (i, 0)),
        ),
        out_shape=jax.ShapeDtypeStruct((BATCH, NUM_LAYERS), jnp.float32),
    )(hard_arr, h_pooled, W, b2)


# speedup vs baseline: 3.9451x; 1.0175x over previous
import functools

import jax
import jax.numpy as jnp
from jax.experimental import pallas as pl
from jax.experimental.pallas import tpu as pltpu

BATCH = 16384
HIDDEN = 4096
NUM_LAYERS = 32
TOPK = 4
BLOCK_R = 1024


def _router_kernel(hard_ref, h_ref, w_ref, b_ref, out_ref):
    logits = jax.lax.dot_general(
        h_ref[...], w_ref[...],
        dimension_numbers=(((1,), (1,)), ((), ())),
        preferred_element_type=jnp.float32,
    ) + b_ref[...]
    sig = jax.nn.sigmoid(logits)

    s = sig
    for _ in range(TOPK - 1):
        m = jnp.max(s, axis=1, keepdims=True)
        s = jnp.where(s == m, -jnp.inf, s)
    thresh = jnp.max(s, axis=1, keepdims=True)
    gates = (sig >= thresh).astype(jnp.float32)

    out_ref[...] = jnp.where(hard_ref[0] != 0, gates, sig)


def kernel(h_pooled, W, b, hard):
    hard_arr = jnp.asarray(hard, dtype=jnp.int32).reshape((1,))
    b2 = b.reshape(1, NUM_LAYERS)
    grid = (BATCH // BLOCK_R,)
    return pl.pallas_call(
        _router_kernel,
        grid_spec=pltpu.PrefetchScalarGridSpec(
            num_scalar_prefetch=1,
            grid=grid,
            in_specs=[
                pl.BlockSpec((BLOCK_R, HIDDEN), lambda i, *_: (i, 0)),
                pl.BlockSpec((NUM_LAYERS, HIDDEN), lambda i, *_: (0, 0)),
                pl.BlockSpec((1, NUM_LAYERS), lambda i, *_: (0, 0)),
            ],
            out_specs=pl.BlockSpec((BLOCK_R, NUM_LAYERS), lambda i, *_: (i, 0)),
        ),
        out_shape=jax.ShapeDtypeStruct((BATCH, NUM_LAYERS), jnp.float32),
    )(hard_arr, h_pooled, W, b2)
